# Initial kernel scaffold; baseline (speedup 1.0000x reference)
import jax
import jax.numpy as jnp
from jax.experimental import pallas as pl


def kernel(reid_features, positions, confs):
    raise NotImplementedError("write your pallas kernel here")


# speedup vs baseline: 3.6905x; 3.6905x over previous
import functools

import jax
import jax.numpy as jnp
import numpy as np
from jax.experimental import pallas as pl

D = 128
K = 16
IMG_W = 1920.0
IMG_H = 1080.0
MAX_DISTANCE = 0.1
THRESH = MAX_DISTANCE * min(IMG_W, IMG_H)
BIG = 1e9


def _acos(x):
    ax = jnp.abs(x)
    p = -0.0012624911
    p = p * ax + 0.0066700901
    p = p * ax - 0.0170881256
    p = p * ax + 0.0308918810
    p = p * ax - 0.0501743046
    p = p * ax + 0.0889789874
    p = p * ax - 0.2145988016
    p = p * ax + 1.5707963050
    r = jnp.sqrt(1.0 - ax) * p
    return jnp.where(x < 0.0, np.pi - r, r)


def _prep_body(reid_ref, pos_ref, f_ref, xywh_ref, posn_ref):
    r = reid_ref[...]
    norm = jnp.sqrt(jnp.sum(r * r, axis=1, keepdims=True))
    f_ref[...] = r / (norm + 1e-12)
    p = pos_ref[...]
    cx = 0.5 * (p[:, 0:1] + p[:, 2:3])
    cy = 0.5 * (p[:, 1:2] + p[:, 3:4])
    w = p[:, 2:3] - p[:, 0:1]
    h = p[:, 3:4] - p[:, 1:2]
    xywh = jnp.concatenate([cx, cy, w, h], axis=1)
    xywh_ref[...] = xywh
    posn_ref[...] = jnp.concatenate(
        [cx / IMG_W, cy / IMG_H, w / IMG_W, h / IMG_H], axis=1
    )


def _main_body(
    f_ref,
    ft_ref,
    cxy_ref,
    xywh_ref,
    topo_ref,
    ang_ref,
    idx_ref,
    xd_ref,
    yd_ref,
    sim_ref,
    *, bq, npad,
):
    pid = pl.program_id(0)
    qx = xywh_ref[:, 0:1]
    qy = xywh_ref[:, 1:2]
    cx = cxy_ref[0:1, :]
    cy = cxy_ref[1:2, :]

    dx = qx - cx
    dy = qy - cy
    dist = jnp.sqrt(dx * dx + dy * dy + 1e-12)

    lane = jax.lax.broadcasted_iota(jnp.int32, (bq, npad), 1)
    row_ids = pid * bq + jax.lax.broadcasted_iota(jnp.int32, (bq, npad), 0)
    not_self = lane != row_ids
    dist = jnp.where((dist <= THRESH) & not_self, dist, BIG)

    sim_full = jax.lax.dot_general(
        f_ref[...], ft_ref[...],
        dimension_numbers=(((1,), (0,)), ((), ())),
        precision=jax.lax.Precision.HIGHEST,
        preferred_element_type=jnp.float32,
    )

    nd_cols = []
    idx_cols = []
    nbx_cols = []
    nby_cols = []
    sim_cols = []
    for _ in range(K):
        minv = jnp.min(dist, axis=1, keepdims=True)
        cand = jnp.where(dist == minv, lane, npad)
        tie = jnp.min(cand, axis=1, keepdims=True)
        onehot = lane == tie
        nbx = jnp.sum(jnp.where(onehot, cx, 0.0), axis=1, keepdims=True)
        nby = jnp.sum(jnp.where(onehot, cy, 0.0), axis=1, keepdims=True)
        simk = jnp.sum(jnp.where(onehot, sim_full, 0.0), axis=1, keepdims=True)
        nd_cols.append(minv)
        idx_cols.append(tie)
        nbx_cols.append(nbx)
        nby_cols.append(nby)
        sim_cols.append(simk)
        dist = jnp.where(onehot, 2.0 * BIG, dist)

    valid_cols = [nd < (BIG * 0.5) for nd in nd_cols]

    topo_ref[...] = jnp.concatenate(
        [jnp.where(v, nd, 0.0) / IMG_H for v, nd in zip(valid_cols, nd_cols)],
        axis=1,
    )
    idx_ref[...] = jnp.concatenate(
        [jnp.where(v, ix, 0) for v, ix in zip(valid_cols, idx_cols)], axis=1
    )

    vx = [nbx - qx for nbx in nbx_cols]
    vy = [nby - qy for nby in nby_cols]
    ang_list = []
    for k in range(K - 1):
        dot = vx[k] * vx[k + 1] + vy[k] * vy[k + 1]
        n1 = jnp.sqrt(vx[k] * vx[k] + vy[k] * vy[k])
        n2 = jnp.sqrt(vx[k + 1] * vx[k + 1] + vy[k + 1] * vy[k + 1])
        cosang = jnp.clip(dot / (n1 * n2 + 1e-9), -1.0 + 1e-6, 1.0 - 1e-6)
        ang = _acos(cosang) * (180.0 / np.pi)
        pv = valid_cols[k] & valid_cols[k + 1]
        ang_list.append(jnp.where(pv, ang, 0.0))
    ang_list.append(jnp.zeros_like(ang_list[0]))
    ang_ref[...] = jnp.concatenate(ang_list, axis=1) / 360.0

    xd_ref[...] = jnp.concatenate(
        [jnp.where(v, (x - qx) / IMG_W, 0.0) for v, x in zip(valid_cols, nbx_cols)],
        axis=1,
    )
    yd_ref[...] = jnp.concatenate(
        [jnp.where(v, (y - qy) / IMG_H, 0.0) for v, y in zip(valid_cols, nby_cols)],
        axis=1,
    )
    sim_ref[...] = jnp.concatenate(
        [jnp.where(v, s, 0.0) for v, s in zip(valid_cols, sim_cols)], axis=1
    )


@jax.jit
def kernel(reid_features, positions, confs):
    n = reid_features.shape[0]
    if n % 200 == 0:
        bq = 200
    else:
        bq = n
    grid = n // bq
    npad = ((n + 127) // 128) * 128

    f_norm, pos_xywh, pos_normed = pl.pallas_call(
        _prep_body,
        grid=(grid,),
        in_specs=[
            pl.BlockSpec((bq, D), lambda i: (i, 0)),
            pl.BlockSpec((bq, 4), lambda i: (i, 0)),
        ],
        out_specs=[
            pl.BlockSpec((bq, D), lambda i: (i, 0)),
            pl.BlockSpec((bq, 4), lambda i: (i, 0)),
            pl.BlockSpec((bq, 4), lambda i: (i, 0)),
        ],
        out_shape=[
            jax.ShapeDtypeStruct((n, D), jnp.float32),
            jax.ShapeDtypeStruct((n, 4), jnp.float32),
            jax.ShapeDtypeStruct((n, 4), jnp.float32),
        ],
    )(reid_features, positions)

    ft = jnp.zeros((D, npad), jnp.float32).at[:, :n].set(f_norm.T)
    cxy = jnp.full((2, npad), 1e9, jnp.float32).at[:, :n].set(pos_xywh[:, :2].T)

    body = functools.partial(_main_body, bq=bq, npad=npad)
    topo_d, angles, idx_safe, xdiff, ydiff, simw = pl.pallas_call(
        body,
        grid=(grid,),
        in_specs=[
            pl.BlockSpec((bq, D), lambda i: (i, 0)),
            pl.BlockSpec((D, npad), lambda i: (0, 0)),
            pl.BlockSpec((2, npad), lambda i: (0, 0)),
            pl.BlockSpec((bq, 4), lambda i: (i, 0)),
        ],
        out_specs=[pl.BlockSpec((bq, K), lambda i: (i, 0))] * 6,
        out_shape=[
            jax.ShapeDtypeStruct((n, K), jnp.float32),
            jax.ShapeDtypeStruct((n, K), jnp.float32),
            jax.ShapeDtypeStruct((n, K), jnp.int32),
            jax.ShapeDtypeStruct((n, K), jnp.float32),
            jax.ShapeDtypeStruct((n, K), jnp.float32),
            jax.ShapeDtypeStruct((n, K), jnp.float32),
        ],
    )(f_norm, ft, cxy, pos_xywh)

    node_feature = jnp.concatenate([f_norm, pos_normed, topo_d, angles], axis=1)

    src = idx_safe.reshape(-1)
    dst = jnp.repeat(jnp.arange(n, dtype=jnp.int32), K)
    edge_index = jnp.stack([src, dst], axis=0)

    edge_feature = jnp.stack(
        [xdiff.reshape(-1), ydiff.reshape(-1), simw.reshape(-1)], axis=1
    )
    return node_feature, edge_index, edge_feature


# speedup vs baseline: 6.6096x; 1.7910x over previous
import functools

import jax
import jax.numpy as jnp
import numpy as np
from jax import lax
from jax.experimental import pallas as pl
from jax.experimental.pallas import tpu as pltpu
from jax.experimental.pallas import tpu_sc as plsc

D = 128
K = 16
IMG_W = 1920.0
IMG_H = 1080.0
MAX_DISTANCE = 0.1
THRESH = MAX_DISTANCE * min(IMG_W, IMG_H)
BIG = 1e9

NUM_WORKERS = 32


def _acos(x):
    ax = jnp.abs(x)
    p = -0.0012624911
    p = p * ax + 0.0066700901
    p = p * ax - 0.0170881256
    p = p * ax + 0.0308918810
    p = p * ax - 0.0501743046
    p = p * ax + 0.0889789874
    p = p * ax - 0.2145988016
    p = p * ax + 1.5707963050
    r = jnp.sqrt(1.0 - ax) * p
    return jnp.where(x < 0.0, np.pi - r, r)


def _prep_body(reid_ref, pos_ref, f_ref, xywh_ref, posn_ref):
    r = reid_ref[...]
    norm = jnp.sqrt(jnp.sum(r * r, axis=1, keepdims=True))
    f_ref[...] = r / (norm + 1e-12)
    p = pos_ref[...]
    cx = 0.5 * (p[:, 0:1] + p[:, 2:3])
    cy = 0.5 * (p[:, 1:2] + p[:, 3:4])
    w = p[:, 2:3] - p[:, 0:1]
    h = p[:, 3:4] - p[:, 1:2]
    xywh_ref[...] = jnp.concatenate([cx, cy, w, h], axis=1)
    posn_ref[...] = jnp.concatenate(
        [cx / IMG_W, cy / IMG_H, w / IMG_W, h / IMG_H], axis=1
    )


def _select_body(cxy_ref, xywh_ref, topo_ref, idx_ref, vm_ref, *, bq, npad):
    pid = pl.program_id(0)
    qx = xywh_ref[:, 0:1]
    qy = xywh_ref[:, 1:2]
    cx = cxy_ref[0:1, :]
    cy = cxy_ref[1:2, :]

    dx = qx - cx
    dy = qy - cy
    dist = jnp.sqrt(dx * dx + dy * dy + 1e-12)

    lane = lax.broadcasted_iota(jnp.int32, (bq, npad), 1)
    row_ids = pid * bq + lax.broadcasted_iota(jnp.int32, (bq, npad), 0)
    dist = jnp.where((dist <= THRESH) & (lane != row_ids), dist, BIG)

    nd_cols = []
    idx_cols = []
    for _ in range(K):
        minv = jnp.min(dist, axis=1, keepdims=True)
        cand = jnp.where(dist == minv, lane, npad)
        tie = jnp.min(cand, axis=1, keepdims=True)
        nd_cols.append(minv)
        idx_cols.append(tie)
        dist = jnp.where(lane == tie, 2.0 * BIG, dist)

    valid_cols = [nd < (BIG * 0.5) for nd in nd_cols]
    topo_ref[...] = jnp.concatenate(
        [jnp.where(v, nd, 0.0) / IMG_H for v, nd in zip(valid_cols, nd_cols)],
        axis=1,
    )
    idx_ref[...] = jnp.concatenate(
        [jnp.where(v, ix, 0) for v, ix in zip(valid_cols, idx_cols)], axis=1
    )
    vm_ref[...] = jnp.concatenate(
        [v.astype(jnp.float32) for v in valid_cols], axis=1
    )


def _sc_gather_body(
    cx_hbm, cy_hbm, idx_hbm, f_hbm,
    nbx_hbm, nby_hbm, sim_hbm,
    cx_v, cy_v, idx_v, fd_v, fg_v, nbx_v, nby_v, sim_v, sem,
    *, rows_per_worker, n_pad,
):
    c = lax.axis_index("c")
    s = lax.axis_index("s")
    wid = s * 2 + c
    base = wid * rows_per_worker

    pltpu.sync_copy(cx_hbm, cx_v)
    pltpu.sync_copy(cy_hbm, cy_v)
    pltpu.sync_copy(idx_hbm.at[pl.ds(base, rows_per_worker), :], idx_v)
    pltpu.sync_copy(f_hbm.at[pl.ds(base, rows_per_worker), :], fd_v)

    lanes = lax.broadcasted_iota(jnp.int32, (K,), 0)

    def row_body(r, carry):
        idxr = idx_v[r, :]
        nbx_v[r, :] = plsc.load_gather(cx_v, [idxr])
        nby_v[r, :] = plsc.load_gather(cy_v, [idxr])
        pltpu.async_copy(f_hbm.at[idxr], fg_v, sem).wait()
        simrow = jnp.zeros((K,), jnp.float32)
        for l in range(K):
            acc = fg_v[l, pl.ds(0, 16)] * fd_v[r, pl.ds(0, 16)]
            for ch in range(1, D // 16):
                acc = acc + fg_v[l, pl.ds(ch * 16, 16)] * fd_v[r, pl.ds(ch * 16, 16)]
            sval = jnp.sum(acc)
            simrow = jnp.where(lanes == l, sval, simrow)
        sim_v[r, :] = simrow
        return carry

    lax.fori_loop(0, rows_per_worker, row_body, 0)

    pltpu.sync_copy(nbx_v, nbx_hbm.at[pl.ds(base, rows_per_worker), :])
    pltpu.sync_copy(nby_v, nby_hbm.at[pl.ds(base, rows_per_worker), :])
    pltpu.sync_copy(sim_v, sim_hbm.at[pl.ds(base, rows_per_worker), :])


def _epilogue_body(
    xywh_ref, nbx_ref, nby_ref, vm_ref, sim_in_ref,
    ang_ref, xd_ref, yd_ref, sim_ref,
):
    qx = xywh_ref[:, 0:1]
    qy = xywh_ref[:, 1:2]
    vm = vm_ref[...]
    vx = nbx_ref[...] - qx
    vy = nby_ref[...] - qy

    n1 = jnp.sqrt(vx * vx + vy * vy)
    ang_cols = []
    for k in range(K - 1):
        dot = vx[:, k:k + 1] * vx[:, k + 1:k + 2] + vy[:, k:k + 1] * vy[:, k + 1:k + 2]
        denom = n1[:, k:k + 1] * n1[:, k + 1:k + 2] + 1e-9
        cosang = jnp.clip(dot / denom, -1.0 + 1e-6, 1.0 - 1e-6)
        ang = _acos(cosang) * (180.0 / np.pi)
        pv = (vm[:, k:k + 1] * vm[:, k + 1:k + 2])
        ang_cols.append(ang * pv)
    ang_cols.append(jnp.zeros_like(ang_cols[0]))
    ang_ref[...] = jnp.concatenate(ang_cols, axis=1) / 360.0

    xd_ref[...] = (vx / IMG_W) * vm
    yd_ref[...] = (vy / IMG_H) * vm
    sim_ref[...] = sim_in_ref[...] * vm


@jax.jit
def kernel(reid_features, positions, confs):
    n = reid_features.shape[0]
    if n % 200 == 0:
        bq = 200
    else:
        bq = n
    grid = n // bq
    npad = ((n + 127) // 128) * 128

    f_norm, pos_xywh, pos_normed = pl.pallas_call(
        _prep_body,
        grid=(grid,),
        in_specs=[
            pl.BlockSpec((bq, D), lambda i: (i, 0)),
            pl.BlockSpec((bq, 4), lambda i: (i, 0)),
        ],
        out_specs=[
            pl.BlockSpec((bq, D), lambda i: (i, 0)),
            pl.BlockSpec((bq, 4), lambda i: (i, 0)),
            pl.BlockSpec((bq, 4), lambda i: (i, 0)),
        ],
        out_shape=[
            jax.ShapeDtypeStruct((n, D), jnp.float32),
            jax.ShapeDtypeStruct((n, 4), jnp.float32),
            jax.ShapeDtypeStruct((n, 4), jnp.float32),
        ],
    )(reid_features, positions)

    cxy = jnp.full((2, npad), 1e9, jnp.float32).at[:, :n].set(pos_xywh[:, :2].T)

    sel = functools.partial(_select_body, bq=bq, npad=npad)
    topo_d, idx_safe, vmask = pl.pallas_call(
        sel,
        grid=(grid,),
        in_specs=[
            pl.BlockSpec((2, npad), lambda i: (0, 0)),
            pl.BlockSpec((bq, 4), lambda i: (i, 0)),
        ],
        out_specs=[
            pl.BlockSpec((bq, K), lambda i: (i, 0)),
            pl.BlockSpec((bq, K), lambda i: (i, 0)),
            pl.BlockSpec((bq, K), lambda i: (i, 0)),
        ],
        out_shape=[
            jax.ShapeDtypeStruct((n, K), jnp.float32),
            jax.ShapeDtypeStruct((n, K), jnp.int32),
            jax.ShapeDtypeStruct((n, K), jnp.float32),
        ],
    )(cxy, pos_xywh)

    n_pad = ((n + 8 * NUM_WORKERS - 1) // (8 * NUM_WORKERS)) * (8 * NUM_WORKERS)
    rpw = n_pad // NUM_WORKERS
    idx_pad = jnp.zeros((n_pad, K), jnp.int32).at[:n].set(idx_safe)
    cx_pad = jnp.zeros((n_pad,), jnp.float32).at[:n].set(pos_xywh[:, 0])
    cy_pad = jnp.zeros((n_pad,), jnp.float32).at[:n].set(pos_xywh[:, 1])
    f_pad = jnp.zeros((n_pad, D), jnp.float32).at[:n].set(f_norm)

    sc_body = functools.partial(
        _sc_gather_body, rows_per_worker=rpw, n_pad=n_pad
    )
    sc_fn = pl.kernel(
        sc_body,
        out_type=[
            jax.ShapeDtypeStruct((n_pad, K), jnp.float32),
            jax.ShapeDtypeStruct((n_pad, K), jnp.float32),
            jax.ShapeDtypeStruct((n_pad, K), jnp.float32),
        ],
        mesh=plsc.VectorSubcoreMesh(core_axis_name="c", subcore_axis_name="s"),
        compiler_params=pltpu.CompilerParams(needs_layout_passes=False),
        scratch_types=[
            pltpu.VMEM((n_pad,), jnp.float32),
            pltpu.VMEM((n_pad,), jnp.float32),
            pltpu.VMEM((rpw, K), jnp.int32),
            pltpu.VMEM((rpw, D), jnp.float32),
            pltpu.VMEM((K, D), jnp.float32),
            pltpu.VMEM((rpw, K), jnp.float32),
            pltpu.VMEM((rpw, K), jnp.float32),
            pltpu.VMEM((rpw, K), jnp.float32),
            pltpu.SemaphoreType.DMA,
        ],
    )
    nbx_p, nby_p, sim_p = sc_fn(cx_pad, cy_pad, idx_pad, f_pad)
    nbx = nbx_p[:n]
    nby = nby_p[:n]
    sim_raw = sim_p[:n]

    angles, xdiff, ydiff, simw = pl.pallas_call(
        _epilogue_body,
        grid=(grid,),
        in_specs=[
            pl.BlockSpec((bq, 4), lambda i: (i, 0)),
            pl.BlockSpec((bq, K), lambda i: (i, 0)),
            pl.BlockSpec((bq, K), lambda i: (i, 0)),
            pl.BlockSpec((bq, K), lambda i: (i, 0)),
            pl.BlockSpec((bq, K), lambda i: (i, 0)),
        ],
        out_specs=[pl.BlockSpec((bq, K), lambda i: (i, 0))] * 4,
        out_shape=[
            jax.ShapeDtypeStruct((n, K), jnp.float32),
            jax.ShapeDtypeStruct((n, K), jnp.float32),
            jax.ShapeDtypeStruct((n, K), jnp.float32),
            jax.ShapeDtypeStruct((n, K), jnp.float32),
        ],
    )(pos_xywh, nbx, nby, vmask, sim_raw)

    node_feature = jnp.concatenate([f_norm, pos_normed, topo_d, angles], axis=1)

    src = idx_safe.reshape(-1)
    dst = jnp.repeat(jnp.arange(n, dtype=jnp.int32), K)
    edge_index = jnp.stack([src, dst], axis=0)

    edge_feature = jnp.stack(
        [xdiff.reshape(-1), ydiff.reshape(-1), simw.reshape(-1)], axis=1
    )
    return node_feature, edge_index, edge_feature


# speedup vs baseline: 7.1785x; 1.0861x over previous
import functools

import jax
import jax.numpy as jnp
import numpy as np
from jax import lax
from jax.experimental import pallas as pl
from jax.experimental.pallas import tpu as pltpu
from jax.experimental.pallas import tpu_sc as plsc

D = 128
K = 16
IMG_W = 1920.0
IMG_H = 1080.0
MAX_DISTANCE = 0.1
THRESH = MAX_DISTANCE * min(IMG_W, IMG_H)
BIG = 1e9

NUM_WORKERS = 32


def _acos(x):
    ax = jnp.abs(x)
    p = -0.0012624911
    p = p * ax + 0.0066700901
    p = p * ax - 0.0170881256
    p = p * ax + 0.0308918810
    p = p * ax - 0.0501743046
    p = p * ax + 0.0889789874
    p = p * ax - 0.2145988016
    p = p * ax + 1.5707963050
    r = jnp.sqrt(1.0 - ax) * p
    return jnp.where(x < 0.0, np.pi - r, r)


def _prep_body(reid_ref, pos_ref, f_ref, xywh_ref, posn_ref):
    r = reid_ref[...]
    norm = jnp.sqrt(jnp.sum(r * r, axis=1, keepdims=True))
    f_ref[...] = r / (norm + 1e-12)
    p = pos_ref[...]
    cx = 0.5 * (p[:, 0:1] + p[:, 2:3])
    cy = 0.5 * (p[:, 1:2] + p[:, 3:4])
    w = p[:, 2:3] - p[:, 0:1]
    h = p[:, 3:4] - p[:, 1:2]
    xywh_ref[...] = jnp.concatenate([cx, cy, w, h], axis=1)
    posn_ref[...] = jnp.concatenate(
        [cx / IMG_W, cy / IMG_H, w / IMG_W, h / IMG_H], axis=1
    )


def _select_body(post_ref, pos_ref, topo_ref, idx_ref, vm_ref, *, bq, npad):
    pid = pl.program_id(0)
    p = pos_ref[...]
    qx = 0.5 * (p[:, 0:1] + p[:, 2:3])
    qy = 0.5 * (p[:, 1:2] + p[:, 3:4])
    cx = 0.5 * (post_ref[0:1, :] + post_ref[2:3, :])
    cy = 0.5 * (post_ref[1:2, :] + post_ref[3:4, :])

    dx = qx - cx
    dy = qy - cy
    dist = jnp.sqrt(dx * dx + dy * dy + 1e-12)

    lane = lax.broadcasted_iota(jnp.int32, (bq, npad), 1)
    row_ids = pid * bq + lax.broadcasted_iota(jnp.int32, (bq, npad), 0)
    dist = jnp.where((dist <= THRESH) & (lane != row_ids), dist, BIG)

    nd_cols = []
    idx_cols = []
    for _ in range(K):
        minv = jnp.min(dist, axis=1, keepdims=True)
        cand = jnp.where(dist == minv, lane, npad)
        tie = jnp.min(cand, axis=1, keepdims=True)
        nd_cols.append(minv)
        idx_cols.append(tie)
        dist = jnp.where(lane == tie, 2.0 * BIG, dist)

    valid_cols = [nd < (BIG * 0.5) for nd in nd_cols]
    topo_ref[...] = jnp.concatenate(
        [jnp.where(v, nd, 0.0) / IMG_H for v, nd in zip(valid_cols, nd_cols)],
        axis=1,
    )
    idx_ref[...] = jnp.concatenate(
        [jnp.where(v, ix, 0) for v, ix in zip(valid_cols, idx_cols)], axis=1
    )
    vm_ref[...] = jnp.concatenate(
        [v.astype(jnp.float32) for v in valid_cols], axis=1
    )


CR = 8


def _sc_gather_body(
    cx_hbm, cy_hbm, idxf_hbm, f_hbm,
    nbx_hbm, nby_hbm, sim_hbm,
    cx_v, cy_v, idxf_v, fd_v, fg_v, nbx_v, nby_v, sim_v, sem,
    *, rows_per_worker,
):
    c = lax.axis_index("c")
    s = lax.axis_index("s")
    wid = s * 2 + c
    base = wid * rows_per_worker

    pltpu.sync_copy(cx_hbm, cx_v)
    pltpu.sync_copy(cy_hbm, cy_v)
    pltpu.sync_copy(idxf_hbm.at[pl.ds(base * K, rows_per_worker * K)], idxf_v)
    pltpu.sync_copy(f_hbm.at[pl.ds(base, rows_per_worker), :], fd_v)

    lanes = lax.broadcasted_iota(jnp.int32, (K,), 0)

    def chunk_body(ci, carry):
        r0 = ci * CR
        pltpu.async_copy(
            f_hbm.at[idxf_v.at[pl.ds(r0 * K, CR * K)]], fg_v, sem
        ).wait()
        for rr in range(CR):
            idxr = idxf_v[pl.ds((r0 + rr) * K, K)]
            nbx_v[r0 + rr, :] = plsc.load_gather(cx_v, [idxr])
            nby_v[r0 + rr, :] = plsc.load_gather(cy_v, [idxr])
            simrow = jnp.zeros((K,), jnp.float32)
            for l in range(K):
                acc = fg_v[rr * K + l, pl.ds(0, 16)] * fd_v[r0 + rr, pl.ds(0, 16)]
                for ch in range(1, D // 16):
                    acc = acc + (
                        fg_v[rr * K + l, pl.ds(ch * 16, 16)]
                        * fd_v[r0 + rr, pl.ds(ch * 16, 16)]
                    )
                sval = jnp.sum(acc)
                simrow = jnp.where(lanes == l, sval, simrow)
            sim_v[r0 + rr, :] = simrow
        return carry

    lax.fori_loop(0, rows_per_worker // CR, chunk_body, 0)

    pltpu.sync_copy(nbx_v, nbx_hbm.at[pl.ds(base, rows_per_worker), :])
    pltpu.sync_copy(nby_v, nby_hbm.at[pl.ds(base, rows_per_worker), :])
    pltpu.sync_copy(sim_v, sim_hbm.at[pl.ds(base, rows_per_worker), :])


def _epilogue_body(
    xywh_ref, nbx_ref, nby_ref, vm_ref, sim_in_ref,
    ang_ref, xd_ref, yd_ref, sim_ref,
):
    qx = xywh_ref[:, 0:1]
    qy = xywh_ref[:, 1:2]
    vm = vm_ref[...]
    vx = nbx_ref[...] - qx
    vy = nby_ref[...] - qy

    n1 = jnp.sqrt(vx * vx + vy * vy)
    ang_cols = []
    for k in range(K - 1):
        dot = vx[:, k:k + 1] * vx[:, k + 1:k + 2] + vy[:, k:k + 1] * vy[:, k + 1:k + 2]
        denom = n1[:, k:k + 1] * n1[:, k + 1:k + 2] + 1e-9
        cosang = jnp.clip(dot / denom, -1.0 + 1e-6, 1.0 - 1e-6)
        ang = _acos(cosang) * (180.0 / np.pi)
        pv = (vm[:, k:k + 1] * vm[:, k + 1:k + 2])
        ang_cols.append(ang * pv)
    ang_cols.append(jnp.zeros_like(ang_cols[0]))
    ang_ref[...] = jnp.concatenate(ang_cols, axis=1) / 360.0

    xd_ref[...] = (vx / IMG_W) * vm
    yd_ref[...] = (vy / IMG_H) * vm
    sim_ref[...] = sim_in_ref[...] * vm


@jax.jit
def kernel(reid_features, positions, confs):
    n = reid_features.shape[0]
    if n % 200 == 0:
        bq = 200
    else:
        bq = n
    grid = n // bq
    npad = ((n + 127) // 128) * 128

    f_norm, pos_xywh, pos_normed = pl.pallas_call(
        _prep_body,
        grid=(grid,),
        in_specs=[
            pl.BlockSpec((bq, D), lambda i: (i, 0)),
            pl.BlockSpec((bq, 4), lambda i: (i, 0)),
        ],
        out_specs=[
            pl.BlockSpec((bq, D), lambda i: (i, 0)),
            pl.BlockSpec((bq, 4), lambda i: (i, 0)),
            pl.BlockSpec((bq, 4), lambda i: (i, 0)),
        ],
        out_shape=[
            jax.ShapeDtypeStruct((n, D), jnp.float32),
            jax.ShapeDtypeStruct((n, 4), jnp.float32),
            jax.ShapeDtypeStruct((n, 4), jnp.float32),
        ],
    )(reid_features, positions)

    post = jnp.full((4, npad), 2e9, jnp.float32).at[:, :n].set(positions.T)

    sel = functools.partial(_select_body, bq=bq, npad=npad)
    topo_d, idx_safe, vmask = pl.pallas_call(
        sel,
        grid=(grid,),
        in_specs=[
            pl.BlockSpec((4, npad), lambda i: (0, 0)),
            pl.BlockSpec((bq, 4), lambda i: (i, 0)),
        ],
        out_specs=[
            pl.BlockSpec((bq, K), lambda i: (i, 0)),
            pl.BlockSpec((bq, K), lambda i: (i, 0)),
            pl.BlockSpec((bq, K), lambda i: (i, 0)),
        ],
        out_shape=[
            jax.ShapeDtypeStruct((n, K), jnp.float32),
            jax.ShapeDtypeStruct((n, K), jnp.int32),
            jax.ShapeDtypeStruct((n, K), jnp.float32),
        ],
    )(post, positions)

    n_pad = ((n + 8 * NUM_WORKERS - 1) // (8 * NUM_WORKERS)) * (8 * NUM_WORKERS)
    rpw = n_pad // NUM_WORKERS
    idx_flat = jnp.zeros((n_pad * K,), jnp.int32).at[: n * K].set(
        idx_safe.reshape(-1)
    )
    cx_pad = jnp.zeros((n_pad,), jnp.float32).at[:n].set(pos_xywh[:, 0])
    cy_pad = jnp.zeros((n_pad,), jnp.float32).at[:n].set(pos_xywh[:, 1])
    f_pad = jnp.zeros((n_pad, D), jnp.float32).at[:n].set(f_norm)

    sc_body = functools.partial(_sc_gather_body, rows_per_worker=rpw)
    sc_fn = pl.kernel(
        sc_body,
        out_type=[
            jax.ShapeDtypeStruct((n_pad, K), jnp.float32),
            jax.ShapeDtypeStruct((n_pad, K), jnp.float32),
            jax.ShapeDtypeStruct((n_pad, K), jnp.float32),
        ],
        mesh=plsc.VectorSubcoreMesh(core_axis_name="c", subcore_axis_name="s"),
        compiler_params=pltpu.CompilerParams(needs_layout_passes=False),
        scratch_types=[
            pltpu.VMEM((n_pad,), jnp.float32),
            pltpu.VMEM((n_pad,), jnp.float32),
            pltpu.VMEM((rpw * K,), jnp.int32),
            pltpu.VMEM((rpw, D), jnp.float32),
            pltpu.VMEM((CR * K, D), jnp.float32),
            pltpu.VMEM((rpw, K), jnp.float32),
            pltpu.VMEM((rpw, K), jnp.float32),
            pltpu.VMEM((rpw, K), jnp.float32),
            pltpu.SemaphoreType.DMA,
        ],
    )
    nbx_p, nby_p, sim_p = sc_fn(cx_pad, cy_pad, idx_flat, f_pad)
    nbx = nbx_p[:n]
    nby = nby_p[:n]
    sim_raw = sim_p[:n]

    angles, xdiff, ydiff, simw = pl.pallas_call(
        _epilogue_body,
        grid=(grid,),
        in_specs=[
            pl.BlockSpec((bq, 4), lambda i: (i, 0)),
            pl.BlockSpec((bq, K), lambda i: (i, 0)),
            pl.BlockSpec((bq, K), lambda i: (i, 0)),
            pl.BlockSpec((bq, K), lambda i: (i, 0)),
            pl.BlockSpec((bq, K), lambda i: (i, 0)),
        ],
        out_specs=[pl.BlockSpec((bq, K), lambda i: (i, 0))] * 4,
        out_shape=[
            jax.ShapeDtypeStruct((n, K), jnp.float32),
            jax.ShapeDtypeStruct((n, K), jnp.float32),
            jax.ShapeDtypeStruct((n, K), jnp.float32),
            jax.ShapeDtypeStruct((n, K), jnp.float32),
        ],
    )(pos_xywh, nbx, nby, vmask, sim_raw)

    node_feature = jnp.concatenate([f_norm, pos_normed, topo_d, angles], axis=1)

    src = idx_safe.reshape(-1)
    dst = jnp.repeat(jnp.arange(n, dtype=jnp.int32), K)
    edge_index = jnp.stack([src, dst], axis=0)

    edge_feature = jnp.stack(
        [xdiff.reshape(-1), ydiff.reshape(-1), simw.reshape(-1)], axis=1
    )
    return node_feature, edge_index, edge_feature
